# Initial kernel scaffold; baseline (speedup 1.0000x reference)
import jax
import jax.numpy as jnp
from jax.experimental import pallas as pl


def kernel(out_vtx, out_pts, vtx_batch, pts_batch, W1, b1, W2, b2, W3, b3, W4, b4, temperature):
    raise NotImplementedError("write your pallas kernel here")


# speedup vs baseline: 1.1488x; 1.1488x over previous
import functools

import jax
import jax.numpy as jnp
from jax import lax
from jax.experimental import pallas as pl
from jax.experimental.pallas import tpu as pltpu
from jax.experimental.pallas import tpu_sc as plsc

N_VTX = 4096
N_PTS = 32768
D = 64
TP = 512
N_TILES = N_PTS // TP

_NEG_INF = float("-inf")


def _norm_rows(x, eps=1e-12):
    n = jnp.sqrt(jnp.sum(x * x, axis=1, keepdims=True))
    return x / jnp.maximum(n, eps)


def _sim_kernel(vtx_ref, pts_ref, vb_ref, pb_ref,
                v_out_ref, p_out_ref, p_pad_ref, max_out_ref, idx_out_ref,
                vn_ref, run_max_ref, run_idx_ref):
    i = pl.program_id(0)

    @pl.when(i == 0)
    def _init():
        vn = _norm_rows(vtx_ref[...])
        vn_ref[...] = vn
        v_out_ref[...] = vn
        run_max_ref[...] = jnp.full((N_VTX, 1), _NEG_INF, jnp.float32)
        run_idx_ref[...] = jnp.zeros((N_VTX, 1), jnp.int32)

    pn = _norm_rows(pts_ref[...])
    p_out_ref[...] = pn
    p_pad_ref[:, :D] = pn
    p_pad_ref[:, D:] = jnp.zeros((TP, D), jnp.float32)

    sim = jnp.dot(vn_ref[...], pn.T, preferred_element_type=jnp.float32)
    mask = vb_ref[...] == pb_ref[...]
    sim_m = jnp.where(mask, sim, _NEG_INF)

    tile_max = jnp.max(sim_m, axis=1, keepdims=True)
    jidx = lax.broadcasted_iota(jnp.int32, (N_VTX, TP), 1)
    tile_arg = jnp.min(jnp.where(sim_m == tile_max, jidx, TP),
                       axis=1, keepdims=True) + i * TP

    upd = tile_max > run_max_ref[...]
    run_max_ref[...] = jnp.where(upd, tile_max, run_max_ref[...])
    run_idx_ref[...] = jnp.where(upd, tile_arg, run_idx_ref[...])

    @pl.when(i == N_TILES - 1)
    def _fin():
        max_out_ref[...] = run_max_ref[...]
        idx_out_ref[...] = run_idx_ref[...]


def _run_sim(out_vtx, out_pts, vb2d, pb2d):
    return pl.pallas_call(
        _sim_kernel,
        grid=(N_TILES,),
        in_specs=[
            pl.BlockSpec((N_VTX, D), lambda i: (0, 0)),
            pl.BlockSpec((TP, D), lambda i: (i, 0)),
            pl.BlockSpec((N_VTX, 1), lambda i: (0, 0)),
            pl.BlockSpec((1, TP), lambda i: (0, i)),
        ],
        out_specs=[
            pl.BlockSpec((N_VTX, D), lambda i: (0, 0)),
            pl.BlockSpec((TP, D), lambda i: (i, 0)),
            pl.BlockSpec((TP, 2 * D), lambda i: (i, 0)),
            pl.BlockSpec((N_VTX, 1), lambda i: (0, 0)),
            pl.BlockSpec((N_VTX, 1), lambda i: (0, 0)),
        ],
        out_shape=[
            jax.ShapeDtypeStruct((N_VTX, D), jnp.float32),
            jax.ShapeDtypeStruct((N_PTS, D), jnp.float32),
            jax.ShapeDtypeStruct((N_PTS, 2 * D), jnp.float32),
            jax.ShapeDtypeStruct((N_VTX, 1), jnp.float32),
            jax.ShapeDtypeStruct((N_VTX, 1), jnp.int32),
        ],
        scratch_shapes=[
            pltpu.VMEM((N_VTX, D), jnp.float32),
            pltpu.VMEM((N_VTX, 1), jnp.float32),
            pltpu.VMEM((N_VTX, 1), jnp.int32),
        ],
    )(out_vtx, out_pts, vb2d, pb2d)


def _make_sc_gather():
    info = plsc.get_sparse_core_info()
    nc, ns = info.num_cores, info.num_subcores
    nw = nc * ns
    b_per_w = N_VTX // nw
    mesh = plsc.VectorSubcoreMesh(core_axis_name="c", subcore_axis_name="s")

    @functools.partial(
        pl.kernel, mesh=mesh,
        out_type=jax.ShapeDtypeStruct((N_VTX, 2 * D), jnp.float32),
        scratch_types=[
            pltpu.VMEM((b_per_w,), jnp.int32),
            pltpu.VMEM((b_per_w, 2 * D), jnp.float32),
            pltpu.SemaphoreType.DMA,
        ],
    )
    def gather(table_hbm, idx_hbm, out_hbm, idx_v, rows_v, sem):
        wid = lax.axis_index("s") * nc + lax.axis_index("c")
        base = wid * b_per_w
        pltpu.sync_copy(idx_hbm.at[pl.ds(base, b_per_w)], idx_v)
        pltpu.async_copy(table_hbm.at[idx_v], rows_v, sem).wait()
        pltpu.sync_copy(rows_v, out_hbm.at[pl.ds(base, b_per_w)])

    return gather


def _mlp_kernel(vn_ref, nn_ref, ms_ref,
                w1v_ref, w1p_ref, w1s_ref, b1_ref,
                w2_ref, b2_ref, w3_ref, b3_ref, w4_ref, b4_ref,
                out_ref):
    h = (jnp.dot(vn_ref[...], w1v_ref[...], preferred_element_type=jnp.float32)
         + jnp.dot(nn_ref[...], w1p_ref[...], preferred_element_type=jnp.float32)
         + ms_ref[...] * w1s_ref[...]
         + b1_ref[...])
    h = jnp.maximum(h, 0.0)
    h = jnp.maximum(jnp.dot(h, w2_ref[...], preferred_element_type=jnp.float32)
                    + b2_ref[...], 0.0)
    h = jnp.maximum(jnp.dot(h, w3_ref[...], preferred_element_type=jnp.float32)
                    + b3_ref[...], 0.0)
    out_ref[...] = (jnp.dot(h, w4_ref[...], preferred_element_type=jnp.float32)
                    + b4_ref[...])


def _run_mlp(vn, nn, ms, W1, b1, W2, b2, W3, b3, W4, b4):
    w1v = W1[:D]
    w1p = jnp.concatenate([W1[D:2 * D], jnp.zeros((D, W1.shape[1]), W1.dtype)],
                          axis=0)
    w1s = W1[2 * D:]
    args = (vn, nn, ms, w1v, w1p, w1s, b1.reshape(1, -1),
            W2, b2.reshape(1, -1), W3, b3.reshape(1, -1),
            W4, b4.reshape(1, -1))
    return pl.pallas_call(
        _mlp_kernel,
        out_shape=jax.ShapeDtypeStruct((N_VTX, 1), jnp.float32),
    )(*args)


def kernel(out_vtx, out_pts, vtx_batch, pts_batch,
           W1, b1, W2, b2, W3, b3, W4, b4, temperature):
    vb2d = vtx_batch.astype(jnp.int32).reshape(N_VTX, 1)
    pb2d = pts_batch.astype(jnp.int32).reshape(1, N_PTS)

    v, p, p_pad, max_sim, nnidx = _run_sim(out_vtx, out_pts, vb2d, pb2d)
    nn_pts = _make_sc_gather()(p_pad, nnidx.reshape(N_VTX))
    out_vismask = _run_mlp(v, nn_pts, max_sim,
                           W1, b1, W2, b2, W3, b3, W4, b4)
    return (v, p, out_vismask, temperature)
